# baseline (speedup 1.0000x reference)
import functools

import jax
import jax.numpy as jnp
from jax.experimental import pallas as pl
from jax.experimental.pallas import tpu as pltpu


def _silu(v):
    return v * jax.nn.sigmoid(v)


def _dot(a, b):
    return jnp.dot(a, b, preferred_element_type=jnp.float32)


def _pre_body(x_ref, rbf_ref, wji_ref, bji_ref, wkj_ref, bkj_ref, wrbf_ref,
              wdown_ref, xji_ref, t_ref):
    x = x_ref[...]
    xji_ref[...] = _silu(_dot(x, wji_ref[...]) + bji_ref[...])
    h_kj = _silu(_dot(x, wkj_ref[...]) + bkj_ref[...])
    h_kj = h_kj * _dot(rbf_ref[...], wrbf_ref[...])
    t_ref[...] = _silu(_dot(h_kj, wdown_ref[...]))


def _stage_a(x, rbf, p, blk=4000):
    E, EMB = x.shape
    NRBF = rbf.shape[1]
    INT = p['W_down'].shape[1]
    w_rbf = _dot(p['W_rbf1'], p['W_rbf2'])
    grid = (E // blk,)
    full = lambda shape: pl.BlockSpec(shape, lambda i: (0, 0))
    return pl.pallas_call(
        _pre_body,
        grid=grid,
        in_specs=[
            pl.BlockSpec((blk, EMB), lambda i: (i, 0)),
            pl.BlockSpec((blk, NRBF), lambda i: (i, 0)),
            full((EMB, EMB)),
            full((1, EMB)),
            full((EMB, EMB)),
            full((1, EMB)),
            full((NRBF, EMB)),
            full((EMB, INT)),
        ],
        out_specs=[
            pl.BlockSpec((blk, EMB), lambda i: (i, 0)),
            pl.BlockSpec((blk, INT), lambda i: (i, 0)),
        ],
        out_shape=[
            jax.ShapeDtypeStruct((E, EMB), jnp.float32),
            jax.ShapeDtypeStruct((E, INT), jnp.float32),
        ],
    )(x, rbf, p['W_ji'], p['b_ji'][None, :], p['W_kj'], p['b_kj'][None, :],
      w_rbf, p['W_down'])


def _sbf_body(sbf_ref, w_ref, out_ref):
    out_ref[...] = _dot(sbf_ref[...], w_ref[...])


def _stage_b(sbf, p, blk=8000):
    A, NSBF = sbf.shape
    INT = p['W_sbf2'].shape[1]
    w_sbf = _dot(p['W_sbf1'], p['W_sbf2'])
    return pl.pallas_call(
        _sbf_body,
        grid=(A // blk,),
        in_specs=[
            pl.BlockSpec((blk, NSBF), lambda i: (i, 0)),
            pl.BlockSpec((NSBF, INT), lambda i: (0, 0)),
        ],
        out_specs=pl.BlockSpec((blk, INT), lambda i: (i, 0)),
        out_shape=jax.ShapeDtypeStruct((A, INT), jnp.float32),
    )(sbf, w_sbf)


def _post_body(pooled_ref, xji_ref, x_ref, wup_ref, wb0_ref, bb0_ref, wb1_ref,
               bb1_ref, wf_ref, bf_ref, wa_ref, ba_ref, out_ref):
    h = xji_ref[...] + _silu(_dot(pooled_ref[...], wup_ref[...]))
    th = _silu(_dot(h, wb0_ref[...]) + bb0_ref[...])
    h = h + _silu(_dot(th, wb1_ref[...]) + bb1_ref[...])
    h = _silu(_dot(h, wf_ref[...]) + bf_ref[...])
    o = x_ref[...] + h
    for i in range(2):
        t0 = _silu(_dot(o, wa_ref[i, 0]) + ba_ref[i, 0][None, :])
        t1 = _silu(_dot(t0, wa_ref[i, 1]) + ba_ref[i, 1][None, :])
        o = o + t1
    out_ref[...] = o


def _stage_d(pooled, x_ji, x, p, blk=4000):
    E, EMB = x.shape
    INT = pooled.shape[1]
    full = lambda shape: pl.BlockSpec(shape, lambda i: tuple(0 for _ in shape))
    return pl.pallas_call(
        _post_body,
        grid=(E // blk,),
        in_specs=[
            pl.BlockSpec((blk, INT), lambda i: (i, 0)),
            pl.BlockSpec((blk, EMB), lambda i: (i, 0)),
            pl.BlockSpec((blk, EMB), lambda i: (i, 0)),
            full((INT, EMB)),
            full((EMB, EMB)),
            full((1, EMB)),
            full((EMB, EMB)),
            full((1, EMB)),
            full((EMB, EMB)),
            full((1, EMB)),
            full((2, 2, EMB, EMB)),
            full((2, 2, EMB)),
        ],
        out_specs=pl.BlockSpec((blk, EMB), lambda i: (i, 0)),
        out_shape=jax.ShapeDtypeStruct((E, EMB), jnp.float32),
    )(pooled, x_ji, x, p['W_up'],
      p['res_before_W'][0, 0], p['res_before_b'][0, 0][None, :],
      p['res_before_W'][0, 1], p['res_before_b'][0, 1][None, :],
      p['W_final'], p['b_final'][None, :],
      p['res_after_W'], p['res_after_b'])


def kernel(x, rbf, sbf, id_expand, params):
    E = x.shape[0]
    x_ji, t = _stage_a(x, rbf, params)
    sbf_e = _stage_b(sbf, params)
    m = jnp.take(t, id_expand[:, 1], axis=0) * sbf_e
    pooled = jax.ops.segment_sum(m, id_expand[:, 0], num_segments=E)
    return _stage_d(pooled, x_ji, x, params)
